# baseline (speedup 1.0000x reference)
import functools

import jax
import jax.numpy as jnp
from jax.experimental import pallas as pl
from jax.experimental.pallas import tpu as pltpu


def _gru_block(h_ref, j_ref, wm_ref, bm_ref, wim_ref, wij_ref, whh_ref,
               bih_ref, bhh_ref, out_ref, *, hid):
    hb = h_ref[...]
    hb16 = hb.astype(jnp.bfloat16)
    msg = jnp.tanh(
        jnp.dot(hb16, wm_ref[...], preferred_element_type=jnp.float32)
        + bm_ref[...])
    gi = (jnp.dot(msg.astype(jnp.bfloat16), wim_ref[...],
                  preferred_element_type=jnp.float32)
          + jnp.dot(j_ref[...].astype(jnp.bfloat16), wij_ref[...],
                    preferred_element_type=jnp.float32)
          + bih_ref[...])
    gh = (jnp.dot(hb16, whh_ref[...], preferred_element_type=jnp.float32)
          + bhh_ref[...])
    i_r, i_z, i_n = gi[:, :hid], gi[:, hid:2 * hid], gi[:, 2 * hid:]
    h_r, h_z, h_n = gh[:, :hid], gh[:, hid:2 * hid], gh[:, 2 * hid:]
    r = jax.nn.sigmoid(i_r + h_r)
    z = jax.nn.sigmoid(i_z + h_z)
    n = jnp.tanh(i_n + r * h_n)
    out_ref[...] = (1.0 - z) * n + z * hb


def kernel(h, jets, mask, W_msg, b_msg, W_ih, W_hh, b_ih, b_hh):
    del mask
    B, N, HID = h.shape
    FEAT = jets.shape[-1]
    M = B * N
    h2 = h.reshape(M, HID)
    j2 = jets.reshape(M, FEAT)

    Wm = W_msg.T.astype(jnp.bfloat16)
    Wim = W_ih[:, :HID].T.astype(jnp.bfloat16)
    Wij = W_ih[:, HID:].T.astype(jnp.bfloat16)
    Whh = W_hh.T.astype(jnp.bfloat16)
    bm = b_msg.reshape(1, HID)
    bih = b_ih.reshape(1, 3 * HID)
    bhh = b_hh.reshape(1, 3 * HID)

    BM = 1024
    grid = (M // BM,)

    row_spec = lambda w: pl.BlockSpec((BM, w), lambda i: (i, 0))
    full_spec = lambda a: pl.BlockSpec(a.shape, lambda i: (0, 0))

    out = pl.pallas_call(
        functools.partial(_gru_block, hid=HID),
        grid=grid,
        in_specs=[
            row_spec(HID),
            row_spec(FEAT),
            full_spec(Wm), full_spec(bm),
            full_spec(Wim), full_spec(Wij),
            full_spec(Whh), full_spec(bih), full_spec(bhh),
        ],
        out_specs=row_spec(HID),
        out_shape=jax.ShapeDtypeStruct((M, HID), jnp.float32),
        compiler_params=pltpu.CompilerParams(
            dimension_semantics=("parallel",),
        ),
    )(h2, j2, Wm, bm, Wim, Wij, Whh, bih, bhh)
    return out.reshape(B, N, HID)


# speedup vs baseline: 1.0495x; 1.0495x over previous
import functools

import jax
import jax.numpy as jnp
from jax.experimental import pallas as pl
from jax.experimental.pallas import tpu as pltpu


def _gru_block(h_ref, j_ref, wm_ref, bm_ref, wim_ref, wij_ref, whh_ref,
               bih_ref, bhh_ref, out_ref, *, hid):
    hb = h_ref[...]
    hb16 = hb.astype(jnp.bfloat16)
    msg = jnp.tanh(
        jnp.dot(hb16, wm_ref[...], preferred_element_type=jnp.float32)
        + bm_ref[...])
    gi = (jnp.dot(msg.astype(jnp.bfloat16), wim_ref[...],
                  preferred_element_type=jnp.float32)
          + jnp.dot(j_ref[...].astype(jnp.bfloat16), wij_ref[...],
                    preferred_element_type=jnp.float32)
          + bih_ref[...])
    gh = (jnp.dot(hb16, whh_ref[...], preferred_element_type=jnp.float32)
          + bhh_ref[...])
    i_r, i_z, i_n = gi[:, :hid], gi[:, hid:2 * hid], gi[:, 2 * hid:]
    h_r, h_z, h_n = gh[:, :hid], gh[:, hid:2 * hid], gh[:, 2 * hid:]
    r = jax.nn.sigmoid(i_r + h_r)
    z = jax.nn.sigmoid(i_z + h_z)
    n = jnp.tanh(i_n + r * h_n)
    out_ref[...] = (1.0 - z) * n + z * hb


def kernel(h, jets, mask, W_msg, b_msg, W_ih, W_hh, b_ih, b_hh):
    del mask
    B, N, HID = h.shape
    FEAT = jets.shape[-1]
    M = B * N
    h2 = h.reshape(M, HID)
    j2 = jets.reshape(M, FEAT)

    Wm = W_msg.T.astype(jnp.bfloat16)
    Wim = W_ih[:, :HID].T.astype(jnp.bfloat16)
    Wij = W_ih[:, HID:].T.astype(jnp.bfloat16)
    Whh = W_hh.T.astype(jnp.bfloat16)
    bm = b_msg.reshape(1, HID)
    bih = b_ih.reshape(1, 3 * HID)
    bhh = b_hh.reshape(1, 3 * HID)

    BM = 2048
    grid = (M // BM,)

    row_spec = lambda w: pl.BlockSpec((BM, w), lambda i: (i, 0))
    full_spec = lambda a: pl.BlockSpec(a.shape, lambda i: (0, 0))

    out = pl.pallas_call(
        functools.partial(_gru_block, hid=HID),
        grid=grid,
        in_specs=[
            row_spec(HID),
            row_spec(FEAT),
            full_spec(Wm), full_spec(bm),
            full_spec(Wim), full_spec(Wij),
            full_spec(Whh), full_spec(bih), full_spec(bhh),
        ],
        out_specs=row_spec(HID),
        out_shape=jax.ShapeDtypeStruct((M, HID), jnp.float32),
        compiler_params=pltpu.CompilerParams(
            dimension_semantics=("parallel",),
        ),
    )(h2, j2, Wm, bm, Wim, Wij, Whh, bih, bhh)
    return out.reshape(B, N, HID)


# speedup vs baseline: 1.0516x; 1.0020x over previous
import functools

import jax
import jax.numpy as jnp
from jax.experimental import pallas as pl
from jax.experimental.pallas import tpu as pltpu


def _gru_block(h_ref, j_ref, wm_ref, bm_ref, wim_ref, wij_ref, whh_ref,
               bih_ref, bhh_ref, out_ref, *, hid):
    hb = h_ref[...]
    hb16 = hb.astype(jnp.bfloat16)
    msg = jnp.tanh(
        jnp.dot(hb16, wm_ref[...], preferred_element_type=jnp.float32)
        + bm_ref[...])
    gi = (jnp.dot(msg.astype(jnp.bfloat16), wim_ref[...],
                  preferred_element_type=jnp.float32)
          + jnp.dot(j_ref[...].astype(jnp.bfloat16), wij_ref[...],
                    preferred_element_type=jnp.float32)
          + bih_ref[...])
    gh = (jnp.dot(hb16, whh_ref[...], preferred_element_type=jnp.float32)
          + bhh_ref[...])
    i_r, i_z, i_n = gi[:, :hid], gi[:, hid:2 * hid], gi[:, 2 * hid:]
    h_r, h_z, h_n = gh[:, :hid], gh[:, hid:2 * hid], gh[:, 2 * hid:]
    r = jax.nn.sigmoid(i_r + h_r)
    z = jax.nn.sigmoid(i_z + h_z)
    n = jnp.tanh(i_n + r * h_n)
    out_ref[...] = (1.0 - z) * n + z * hb


def kernel(h, jets, mask, W_msg, b_msg, W_ih, W_hh, b_ih, b_hh):
    del mask
    B, N, HID = h.shape
    FEAT = jets.shape[-1]
    M = B * N
    h2 = h.reshape(M, HID)
    j2 = jets.reshape(M, FEAT)

    Wm = W_msg.T.astype(jnp.bfloat16)
    Wim = W_ih[:, :HID].T.astype(jnp.bfloat16)
    Wij = W_ih[:, HID:].T.astype(jnp.bfloat16)
    Whh = W_hh.T.astype(jnp.bfloat16)
    bm = b_msg.reshape(1, HID)
    bih = b_ih.reshape(1, 3 * HID)
    bhh = b_hh.reshape(1, 3 * HID)

    BM = 4096
    grid = (M // BM,)

    row_spec = lambda w: pl.BlockSpec((BM, w), lambda i: (i, 0))
    full_spec = lambda a: pl.BlockSpec(a.shape, lambda i: (0, 0))

    out = pl.pallas_call(
        functools.partial(_gru_block, hid=HID),
        grid=grid,
        in_specs=[
            row_spec(HID),
            row_spec(FEAT),
            full_spec(Wm), full_spec(bm),
            full_spec(Wim), full_spec(Wij),
            full_spec(Whh), full_spec(bih), full_spec(bhh),
        ],
        out_specs=row_spec(HID),
        out_shape=jax.ShapeDtypeStruct((M, HID), jnp.float32),
        compiler_params=pltpu.CompilerParams(
            dimension_semantics=("parallel",),
        ),
    )(h2, j2, Wm, bm, Wim, Wij, Whh, bih, bhh)
    return out.reshape(B, N, HID)


# speedup vs baseline: 1.0600x; 1.0079x over previous
import functools

import jax
import jax.numpy as jnp
from jax.experimental import pallas as pl
from jax.experimental.pallas import tpu as pltpu


def _gru_block(h_ref, j_ref, wm_ref, bm_ref, w1_ref, w2_ref, w3_ref,
               bhn_ref, out_ref, x_ref, *, hid, feat):
    kx = x_ref.shape[-1]
    hb = h_ref[...]
    hb16 = hb.astype(jnp.bfloat16)
    x_ref[:, :hid] = hb16
    x_ref[:, 2 * hid:2 * hid + feat] = j_ref[...].astype(jnp.bfloat16)

    @pl.when(pl.program_id(0) == 0)
    def _init_ones():
        col = jax.lax.broadcasted_iota(jnp.int32, (x_ref.shape[0], 8), 1)
        x_ref[:, 2 * hid + feat:] = (col == 0).astype(jnp.bfloat16)

    msg = jnp.tanh(
        jnp.dot(hb16, wm_ref[...], preferred_element_type=jnp.float32)
        + bm_ref[...])
    x_ref[:, hid:2 * hid] = msg.astype(jnp.bfloat16)

    xc = x_ref[...]
    s_rz = jnp.dot(xc, w1_ref[...], preferred_element_type=jnp.float32)
    i_n = jnp.dot(xc[:, hid:], w2_ref[...], preferred_element_type=jnp.float32)
    h_n = (jnp.dot(hb16, w3_ref[...], preferred_element_type=jnp.float32)
           + bhn_ref[...])
    r = jax.nn.sigmoid(s_rz[:, :hid])
    z = jax.nn.sigmoid(s_rz[:, hid:])
    n = jnp.tanh(i_n + r * h_n)
    out_ref[...] = n + z * (hb - n)


def kernel(h, jets, mask, W_msg, b_msg, W_ih, W_hh, b_ih, b_hh):
    del mask
    B, N, HID = h.shape
    FEAT = jets.shape[-1]
    M = B * N
    h2 = h.reshape(M, HID)
    j2 = jets.reshape(M, FEAT)

    f16 = jnp.bfloat16
    Wm = W_msg.T.astype(f16)
    bm = b_msg.reshape(1, HID)
    W1 = jnp.concatenate([
        W_hh[:2 * HID, :].T,
        W_ih[:2 * HID, :HID].T,
        W_ih[:2 * HID, HID:].T,
        (b_ih[:2 * HID] + b_hh[:2 * HID]).reshape(1, 2 * HID),
        jnp.zeros((7, 2 * HID), jnp.float32),
    ], axis=0).astype(f16)
    W2 = jnp.concatenate([
        W_ih[2 * HID:, :HID].T,
        W_ih[2 * HID:, HID:].T,
        b_ih[2 * HID:].reshape(1, HID),
        jnp.zeros((7, HID), jnp.float32),
    ], axis=0).astype(f16)
    W3 = W_hh[2 * HID:, :].T.astype(f16)
    bhn = b_hh[2 * HID:].reshape(1, HID)

    BM = 2048
    KX = 2 * HID + FEAT + 8
    grid = (M // BM,)

    row_spec = lambda w: pl.BlockSpec((BM, w), lambda i: (i, 0))
    full_spec = lambda a: pl.BlockSpec(a.shape, lambda i: (0, 0))

    out = pl.pallas_call(
        functools.partial(_gru_block, hid=HID, feat=FEAT),
        grid=grid,
        in_specs=[
            row_spec(HID),
            row_spec(FEAT),
            full_spec(Wm), full_spec(bm),
            full_spec(W1), full_spec(W2), full_spec(W3), full_spec(bhn),
        ],
        out_specs=row_spec(HID),
        out_shape=jax.ShapeDtypeStruct((M, HID), jnp.float32),
        scratch_shapes=[pltpu.VMEM((BM, KX), f16)],
        compiler_params=pltpu.CompilerParams(
            dimension_semantics=("arbitrary",),
        ),
    )(h2, j2, Wm, bm, W1, W2, W3, bhn)
    return out.reshape(B, N, HID)


# speedup vs baseline: 1.1201x; 1.0567x over previous
import functools

import jax
import jax.numpy as jnp
from jax.experimental import pallas as pl
from jax.experimental.pallas import tpu as pltpu


def _gru_block(h_ref, j_ref, wm_ref, bm_ref, w1_ref, w2_ref, w3_ref,
               bhn_ref, out_ref, x0_ref, x1_ref, *, hid, feat):
    scratches = (x0_ref, x1_ref)
    T = x0_ref.shape[0]

    @pl.when(pl.program_id(0) == 0)
    def _init_ones():
        col = jax.lax.broadcasted_iota(jnp.int32, (T, 8), 1)
        ones = (col == 0).astype(jnp.bfloat16)
        x0_ref[:, 2 * hid + feat:] = ones
        x1_ref[:, 2 * hid + feat:] = ones

    for t, x_ref in enumerate(scratches):
        rows = pl.ds(t * T, T)
        hb = h_ref[rows, :]
        hb16 = hb.astype(jnp.bfloat16)
        x_ref[:, :hid] = hb16
        x_ref[:, 2 * hid:2 * hid + feat] = j_ref[rows, :].astype(jnp.bfloat16)
        msg = jnp.tanh(
            jnp.dot(hb16, wm_ref[...], preferred_element_type=jnp.float32)
            + bm_ref[...])
        x_ref[:, hid:2 * hid] = msg.astype(jnp.bfloat16)

        xc = x_ref[...]
        s_rz = jnp.dot(xc, w1_ref[...], preferred_element_type=jnp.float32)
        i_n = jnp.dot(xc[:, hid:], w2_ref[...],
                      preferred_element_type=jnp.float32)
        h_n = (jnp.dot(hb16, w3_ref[...], preferred_element_type=jnp.float32)
               + bhn_ref[...])
        r = jax.nn.sigmoid(s_rz[:, :hid])
        z = jax.nn.sigmoid(s_rz[:, hid:])
        n = jnp.tanh(i_n + r * h_n)
        out_ref[rows, :] = n + z * (hb - n)


def kernel(h, jets, mask, W_msg, b_msg, W_ih, W_hh, b_ih, b_hh):
    del mask
    B, N, HID = h.shape
    FEAT = jets.shape[-1]
    M = B * N
    h2 = h.reshape(M, HID)
    j2 = jets.reshape(M, FEAT)

    f16 = jnp.bfloat16
    Wm = W_msg.T.astype(f16)
    bm = b_msg.reshape(1, HID)
    W1 = jnp.concatenate([
        W_hh[:2 * HID, :].T,
        W_ih[:2 * HID, :HID].T,
        W_ih[:2 * HID, HID:].T,
        (b_ih[:2 * HID] + b_hh[:2 * HID]).reshape(1, 2 * HID),
        jnp.zeros((7, 2 * HID), jnp.float32),
    ], axis=0).astype(f16)
    W2 = jnp.concatenate([
        W_ih[2 * HID:, :HID].T,
        W_ih[2 * HID:, HID:].T,
        b_ih[2 * HID:].reshape(1, HID),
        jnp.zeros((7, HID), jnp.float32),
    ], axis=0).astype(f16)
    W3 = W_hh[2 * HID:, :].T.astype(f16)
    bhn = b_hh[2 * HID:].reshape(1, HID)

    BM = 2048
    KX = 2 * HID + FEAT + 8
    grid = (M // BM,)

    row_spec = lambda w: pl.BlockSpec((BM, w), lambda i: (i, 0))
    full_spec = lambda a: pl.BlockSpec(a.shape, lambda i: (0, 0))

    out = pl.pallas_call(
        functools.partial(_gru_block, hid=HID, feat=FEAT),
        grid=grid,
        in_specs=[
            row_spec(HID),
            row_spec(FEAT),
            full_spec(Wm), full_spec(bm),
            full_spec(W1), full_spec(W2), full_spec(W3), full_spec(bhn),
        ],
        out_specs=row_spec(HID),
        out_shape=jax.ShapeDtypeStruct((M, HID), jnp.float32),
        scratch_shapes=[pltpu.VMEM((BM // 2, KX), f16),
                        pltpu.VMEM((BM // 2, KX), f16)],
        compiler_params=pltpu.CompilerParams(
            dimension_semantics=("arbitrary",),
        ),
    )(h2, j2, Wm, bm, W1, W2, W3, bhn)
    return out.reshape(B, N, HID)


# speedup vs baseline: 1.1550x; 1.0311x over previous
import functools

import jax
import jax.numpy as jnp
from jax.experimental import pallas as pl
from jax.experimental.pallas import tpu as pltpu


def _gru_block(h_ref, j_ref, wm_ref, bm_ref, w1_ref, w2_ref, w3_ref,
               bhn_ref, out_ref, *x_refs, hid, feat):
    scratches = x_refs
    T = x_refs[0].shape[0]

    @pl.when(pl.program_id(0) == 0)
    def _init_ones():
        col = jax.lax.broadcasted_iota(jnp.int32, (T, 8), 1)
        ones = (col == 0).astype(jnp.bfloat16)
        for x_ref in scratches:
            x_ref[:, 2 * hid + feat:] = ones

    for t, x_ref in enumerate(scratches):
        rows = pl.ds(t * T, T)
        hb = h_ref[rows, :]
        hb16 = hb.astype(jnp.bfloat16)
        x_ref[:, :hid] = hb16
        x_ref[:, 2 * hid:2 * hid + feat] = j_ref[rows, :].astype(jnp.bfloat16)
        msg = jnp.tanh(
            jnp.dot(hb16, wm_ref[...], preferred_element_type=jnp.float32)
            + bm_ref[...])
        x_ref[:, hid:2 * hid] = msg.astype(jnp.bfloat16)

        xc = x_ref[...]
        s_rz = jnp.dot(xc, w1_ref[...], preferred_element_type=jnp.float32)
        i_n = jnp.dot(xc[:, hid:], w2_ref[...],
                      preferred_element_type=jnp.float32)
        h_n = (jnp.dot(hb16, w3_ref[...], preferred_element_type=jnp.float32)
               + bhn_ref[...])
        r = jax.nn.sigmoid(s_rz[:, :hid])
        z = jax.nn.sigmoid(s_rz[:, hid:])
        n = jnp.tanh(i_n + r * h_n)
        out_ref[rows, :] = n + z * (hb - n)


def kernel(h, jets, mask, W_msg, b_msg, W_ih, W_hh, b_ih, b_hh):
    del mask
    B, N, HID = h.shape
    FEAT = jets.shape[-1]
    M = B * N
    h2 = h.reshape(M, HID)
    j2 = jets.reshape(M, FEAT)

    f16 = jnp.bfloat16
    Wm = W_msg.T.astype(f16)
    bm = b_msg.reshape(1, HID)
    W1 = jnp.concatenate([
        W_hh[:2 * HID, :].T,
        W_ih[:2 * HID, :HID].T,
        W_ih[:2 * HID, HID:].T,
        (b_ih[:2 * HID] + b_hh[:2 * HID]).reshape(1, 2 * HID),
        jnp.zeros((7, 2 * HID), jnp.float32),
    ], axis=0).astype(f16)
    W2 = jnp.concatenate([
        W_ih[2 * HID:, :HID].T,
        W_ih[2 * HID:, HID:].T,
        b_ih[2 * HID:].reshape(1, HID),
        jnp.zeros((7, HID), jnp.float32),
    ], axis=0).astype(f16)
    W3 = W_hh[2 * HID:, :].T.astype(f16)
    bhn = b_hh[2 * HID:].reshape(1, HID)

    BM = 8192
    NT = 8
    KX = 2 * HID + FEAT + 8
    grid = (M // BM,)

    row_spec = lambda w: pl.BlockSpec((BM, w), lambda i: (i, 0))
    full_spec = lambda a: pl.BlockSpec(a.shape, lambda i: (0, 0))

    out = pl.pallas_call(
        functools.partial(_gru_block, hid=HID, feat=FEAT),
        grid=grid,
        in_specs=[
            row_spec(HID),
            row_spec(FEAT),
            full_spec(Wm), full_spec(bm),
            full_spec(W1), full_spec(W2), full_spec(W3), full_spec(bhn),
        ],
        out_specs=row_spec(HID),
        out_shape=jax.ShapeDtypeStruct((M, HID), jnp.float32),
        scratch_shapes=[pltpu.VMEM((BM // NT, KX), f16)
                        for _ in range(NT)],
        compiler_params=pltpu.CompilerParams(
            dimension_semantics=("arbitrary",),
        ),
    )(h2, j2, Wm, bm, W1, W2, W3, bhn)
    return out.reshape(B, N, HID)
